# Initial kernel scaffold; baseline (speedup 1.0000x reference)
import jax
import jax.numpy as jnp
from jax.experimental import pallas as pl


def kernel(x, task_ids, module_logits):
    raise NotImplementedError("write your pallas kernel here")


# speedup vs baseline: 1.5235x; 1.5235x over previous
import functools

import jax
import jax.numpy as jnp
from jax import lax
from jax.experimental import pallas as pl
from jax.experimental.pallas import tpu as pltpu
from jax.experimental.pallas import tpu_sc as plsc

_EPS = 1e-08
_N_SPLITS = 8
_N_EXPERTS = 64


def _norm_body(logits_ref, out_ref):
    p = jax.nn.sigmoid(logits_ref[...])
    s = jnp.sum(p, axis=1, keepdims=True)
    out_ref[...] = p / (s + _EPS)


def _normalize_table(module_logits):
    n_tasks, d = module_logits.shape
    rows = n_tasks * _N_SPLITS
    flat = module_logits.reshape(rows, _N_EXPERTS)
    out = pl.pallas_call(
        _norm_body,
        out_shape=jax.ShapeDtypeStruct((rows, _N_EXPERTS), jnp.float32),
    )(flat)
    return out.reshape(n_tasks, d)


def _make_gather(v, d, b):
    info = plsc.get_sparse_core_info()
    nw = info.num_cores * info.num_subcores
    b_per_w = b // nw
    chunk = 128
    n_chunks = b_per_w // chunk
    mesh = plsc.VectorSubcoreMesh(core_axis_name="c", subcore_axis_name="s")

    @functools.partial(
        pl.kernel,
        out_type=jax.ShapeDtypeStruct((b, d), jnp.float32),
        mesh=mesh,
        scratch_types=[
            pltpu.VMEM((chunk,), jnp.int32),
            pltpu.VMEM((chunk, d), jnp.float32),
            pltpu.SemaphoreType.DMA,
        ],
    )
    def gather(table_hbm, idx_hbm, out_hbm, idx_v, rows_v, sem):
        wid = lax.axis_index("s") * info.num_cores + lax.axis_index("c")
        base = wid * b_per_w
        for c in range(n_chunks):
            off = base + c * chunk
            pltpu.sync_copy(idx_hbm.at[pl.ds(off, chunk)], idx_v)
            pltpu.async_copy(table_hbm.at[idx_v], rows_v, sem).wait()
            pltpu.sync_copy(rows_v, out_hbm.at[pl.ds(off, chunk)])

    return gather


def kernel(x, task_ids, module_logits):
    n_tokens = task_ids.shape[0]
    table = _normalize_table(module_logits)
    idx = task_ids.astype(jnp.int32)
    out = _make_gather(table.shape[0], table.shape[1], n_tokens)(table, idx)
    return out.reshape(n_tokens, _N_SPLITS, _N_EXPERTS)


# speedup vs baseline: 1.6369x; 1.0745x over previous
import functools

import jax
import jax.numpy as jnp
from jax import lax
from jax.experimental import pallas as pl
from jax.experimental.pallas import tpu as pltpu
from jax.experimental.pallas import tpu_sc as plsc

_EPS = 1e-08
_N_SPLITS = 8
_N_EXPERTS = 64
_LANES = 16


def _make_fused(v, d, b):
    info = plsc.get_sparse_core_info()
    nc, ns = info.num_cores, info.num_subcores
    nw = nc * ns
    b_per_w = b // nw
    chunk = 64
    n_chunks = b_per_w // chunk
    nb = 2
    rows_per_tile = v // ns
    nq = d // _LANES
    mesh = plsc.VectorSubcoreMesh(core_axis_name="c", subcore_axis_name="s")

    @functools.partial(
        pl.kernel,
        out_type=(
            jax.ShapeDtypeStruct((b, d), jnp.float32),
            jax.ShapeDtypeStruct((nc, v, d), jnp.float32),
        ),
        mesh=mesh,
        scratch_types=[
            pltpu.VMEM((rows_per_tile, d), jnp.float32),
            pltpu.VMEM((n_chunks, chunk), jnp.int32),
            pltpu.VMEM((nb, chunk, d), jnp.float32),
        ]
        + [pltpu.SemaphoreType.DMA] * (2 * nb),
    )
    def fused(logits_hbm, idx_hbm, out_hbm, tables_hbm, slab, idx_v, rows_v, *sems):
        gsems, ssems = sems[:nb], sems[nb:]
        cid = lax.axis_index("c")
        sid = lax.axis_index("s")
        wid = sid * nc + cid

        row0 = sid * rows_per_tile
        pltpu.sync_copy(logits_hbm.at[pl.ds(row0, rows_per_tile)], slab)

        lane = lax.iota(jnp.int32, _LANES)
        dnums = lax.GatherDimensionNumbers(
            offset_dims=(), collapsed_slice_dims=(0,), start_index_map=(0,)
        )

        def take16(vv, idx):
            return lax.gather(
                vv,
                idx[:, None],
                dnums,
                (1,),
                mode=lax.GatherScatterMode.PROMISE_IN_BOUNDS,
            )

        def hsum(vv):
            for st in (1, 2, 4, 8):
                vv = vv + take16(vv, lane ^ st)
            return vv

        def norm_row(r, carry):
            for k in range(_N_SPLITS):
                base_q = k * _N_EXPERTS
                ps = []
                for q in range(_N_EXPERTS // _LANES):
                    x = slab[r, pl.ds(base_q + q * _LANES, _LANES)]
                    ps.append(1.0 / (1.0 + jnp.exp(-x)))
                tot = hsum(ps[0] + ps[1] + ps[2] + ps[3]) + _EPS
                for q in range(_N_EXPERTS // _LANES):
                    slab[r, pl.ds(base_q + q * _LANES, _LANES)] = ps[q] / tot
            return carry

        lax.fori_loop(0, rows_per_tile, norm_row, 0, unroll=False)
        table_c = tables_hbm.at[cid]
        pltpu.sync_copy(slab, table_c.at[pl.ds(row0, rows_per_tile)])
        plsc.subcore_barrier()

        base = wid * b_per_w
        pltpu.sync_copy(idx_hbm.at[pl.ds(wid * n_chunks, n_chunks)], idx_v)
        gh = [None] * nb
        sh = [None] * nb
        for c in range(min(nb, n_chunks)):
            gh[c] = pltpu.async_copy(
                table_c.at[idx_v.at[c]], rows_v.at[c], gsems[c]
            )
        for c in range(n_chunks):
            bslot = c % nb
            gh[bslot].wait()
            sh[bslot] = pltpu.async_copy(
                rows_v.at[bslot],
                out_hbm.at[pl.ds(base + c * chunk, chunk)],
                ssems[bslot],
            )
            nxt = c + nb
            if nxt < n_chunks:
                sh[bslot].wait()
                gh[bslot] = pltpu.async_copy(
                    table_c.at[idx_v.at[nxt]], rows_v.at[bslot], gsems[bslot]
                )
        for c in range(max(0, n_chunks - nb), n_chunks):
            bslot = c % nb
            if sh[bslot] is not None:
                sh[bslot].wait()
                sh[bslot] = None

    return fused


def kernel(x, task_ids, module_logits):
    n_tokens = task_ids.shape[0]
    v, d = module_logits.shape
    chunk = 64
    idx = task_ids.astype(jnp.int32).reshape(n_tokens // chunk, chunk)
    out, _ = _make_fused(v, d, n_tokens)(module_logits, idx)
    return out.reshape(n_tokens, _N_SPLITS, _N_EXPERTS)
